# Initial kernel scaffold; baseline (speedup 1.0000x reference)
import jax
import jax.numpy as jnp
from jax.experimental import pallas as pl


def kernel(src_embedding, edge_index, edge_embedding, W_fc, b_fc, q, Wk, Wv, Wo, Wg, We):
    raise NotImplementedError("write your pallas kernel here")


# speedup vs baseline: 50.2644x; 50.2644x over previous
import math

import jax
import jax.numpy as jnp
from jax import lax
from jax.experimental import pallas as pl
from jax.experimental.pallas import tpu as pltpu
from jax.experimental.pallas import tpu_sc as plsc

N = 10000
E = 320000
D = 128
T = 64
H = 4
DH = 16
NE = 4

NC = 2
NS = 16
NW = NC * NS

CB = 128
NCHUNK = E // CB
ZW = 128

BE = 1600
BN = 1000


def _gather_body(table_hbm, src_hbm, out_hbm, idx_v, rows_v, sem):
    cid = lax.axis_index("c")
    sid = lax.axis_index("s")
    wid = sid * NC + cid
    ncw = 78 + jnp.where(wid < 4, 1, 0)
    start = 78 * wid + jnp.minimum(wid, 4)

    def chunk(i, carry):
        @pl.when(i < ncw)
        def _():
            base = (start + i) * CB
            pltpu.sync_copy(src_hbm.at[pl.ds(base, CB)], idx_v)
            pltpu.async_copy(table_hbm.at[idx_v], rows_v, sem).wait()
            pltpu.sync_copy(rows_v, out_hbm.at[pl.ds(base, CB)])
        return carry

    lax.fori_loop(0, 79, chunk, 0)


def _sc_gather(table, src):
    mesh = plsc.VectorSubcoreMesh(core_axis_name="c", subcore_axis_name="s")
    f = pl.kernel(
        _gather_body,
        out_type=jax.ShapeDtypeStruct((E, D), jnp.float32),
        mesh=mesh,
        scratch_types=[
            pltpu.VMEM((CB,), jnp.int32),
            pltpu.VMEM((CB, D), jnp.float32),
            pltpu.SemaphoreType.DMA,
        ],
    )
    return f(table, src)


def _edge_body(g_ref, e_ref, wfc_ref, bfc_ref, wv_ref, a_ref, r_ref, s64_ref,
               z_ref):
    msg = jnp.maximum(g_ref[...] + e_ref[...], 0.0)
    p = jnp.dot(msg, wfc_ref[...], preferred_element_type=jnp.float32)
    p = p + bfc_ref[...]
    v = jnp.dot(p, wv_ref[...], preferred_element_type=jnp.float32)
    s = jnp.clip(jnp.dot(p, a_ref[...], preferred_element_type=jnp.float32),
                 -60.0, 60.0)
    ex = jnp.exp(s)
    exrep = jnp.dot(ex, r_ref[...], preferred_element_type=jnp.float32)
    vex = v * exrep
    extail = jnp.dot(ex, s64_ref[...], preferred_element_type=jnp.float32)
    z_ref[...] = jnp.concatenate([vex, extail], axis=1)


def _tc_edge(gathered, edge_emb, wfc, bfc, wv, a, r, s64):
    return pl.pallas_call(
        _edge_body,
        grid=(E // BE,),
        in_specs=[
            pl.BlockSpec((BE, D), lambda i: (i, 0)),
            pl.BlockSpec((BE, D), lambda i: (i, 0)),
            pl.BlockSpec((D, T), lambda i: (0, 0)),
            pl.BlockSpec((1, T), lambda i: (0, 0)),
            pl.BlockSpec((T, T), lambda i: (0, 0)),
            pl.BlockSpec((T, H), lambda i: (0, 0)),
            pl.BlockSpec((H, T), lambda i: (0, 0)),
            pl.BlockSpec((H, T), lambda i: (0, 0)),
        ],
        out_specs=pl.BlockSpec((BE, ZW), lambda i: (i, 0)),
        out_shape=jax.ShapeDtypeStruct((E, ZW), jnp.float32),
    )(gathered, edge_emb, wfc, bfc, wv, a, r, s64)


def _scatter_body(z_hbm, dst_hbm, zz_hbm, out0_hbm, out1_hbm,
                  idx_v, z_v, acc_sh):
    cid = lax.axis_index("c")
    sid = lax.axis_index("s")
    wid = sid * NC + cid

    @pl.when(sid == 0)
    def _():
        pltpu.sync_copy(zz_hbm, acc_sh)

    plsc.subcore_barrier()

    ncw = 78 + jnp.where(wid < 4, 1, 0)
    start = 78 * wid + jnp.minimum(wid, 4)

    def chunk(i, carry):
        @pl.when(i < ncw)
        def _():
            base = (start + i) * CB
            pltpu.sync_copy(dst_hbm.at[pl.ds(base, CB)], idx_v)
            pltpu.sync_copy(z_hbm.at[pl.ds(base, CB)], z_v)
            pltpu.sync_copy(z_v, acc_sh.at[idx_v], add=True)
        return carry

    lax.fori_loop(0, 79, chunk, 0)

    plsc.subcore_barrier()

    @pl.when(jnp.logical_and(sid == 0, cid == 0))
    def _():
        pltpu.sync_copy(acc_sh, out0_hbm)

    @pl.when(jnp.logical_and(sid == 0, cid == 1))
    def _():
        pltpu.sync_copy(acc_sh, out1_hbm)


def _sc_scatter(z, dst, zz):
    mesh = plsc.VectorSubcoreMesh(core_axis_name="c", subcore_axis_name="s")
    f = pl.kernel(
        _scatter_body,
        out_type=(jax.ShapeDtypeStruct((N, ZW), jnp.float32),
                  jax.ShapeDtypeStruct((N, ZW), jnp.float32)),
        mesh=mesh,
        scratch_types=[
            pltpu.VMEM((CB,), jnp.int32),
            pltpu.VMEM((CB, ZW), jnp.float32),
            pltpu.VMEM_SHARED((N, ZW), jnp.float32),
        ],
    )
    return f(z, dst, zz)


def _node_body(p0_ref, p1_ref, m_ref, wo_ref, wg_ref, wec_ref, r4_ref, k4_ref,
               out_ref):
    acc = p0_ref[...] + p1_ref[...]
    denrep = jnp.dot(acc, m_ref[...], preferred_element_type=jnp.float32)
    vex = acc[:, :T]
    agg = vex / (denrep + 1e-9)
    attn = jnp.dot(agg, wo_ref[...], preferred_element_type=jnp.float32)
    gl = jnp.dot(attn, wg_ref[...], preferred_element_type=jnp.float32)
    gm = jnp.max(gl, axis=1, keepdims=True)
    ge = jnp.exp(gl - gm)
    gate = ge / jnp.sum(ge, axis=1, keepdims=True)
    expf = jnp.dot(attn, wec_ref[...], preferred_element_type=jnp.float32)
    grep = jnp.dot(gate, r4_ref[...], preferred_element_type=jnp.float32)
    moe = jnp.dot(grep * expf, k4_ref[...], preferred_element_type=jnp.float32)
    out_ref[...] = 1.0 / (1.0 + jnp.exp(-moe))


def _tc_node(p0, p1, m, wo, wg, wec, r4, k4):
    return pl.pallas_call(
        _node_body,
        grid=(N // BN,),
        in_specs=[
            pl.BlockSpec((BN, ZW), lambda i: (i, 0)),
            pl.BlockSpec((BN, ZW), lambda i: (i, 0)),
            pl.BlockSpec((ZW, T), lambda i: (0, 0)),
            pl.BlockSpec((T, T), lambda i: (0, 0)),
            pl.BlockSpec((T, NE), lambda i: (0, 0)),
            pl.BlockSpec((T, NE * T), lambda i: (0, 0)),
            pl.BlockSpec((NE, NE * T), lambda i: (0, 0)),
            pl.BlockSpec((NE * T, T), lambda i: (0, 0)),
        ],
        out_specs=pl.BlockSpec((BN, T), lambda i: (i, 0)),
        out_shape=jax.ShapeDtypeStruct((N, T), jnp.float32),
    )(p0, p1, m, wo, wg, wec, r4, k4)


def kernel(src_embedding, edge_index, edge_embedding, W_fc, b_fc, q, Wk, Wv,
           Wo, Wg, We):
    f32 = jnp.float32
    src = edge_index[0].astype(jnp.int32)
    dst = edge_index[1].astype(jnp.int32)

    A = (Wk.reshape(T, H, DH) * q[None, :, :]).sum(-1) * (1.0 / math.sqrt(DH))
    R = jnp.kron(jnp.eye(H, dtype=f32), jnp.ones((1, DH), f32))
    S64 = jnp.eye(H, T, dtype=f32)
    M = jnp.concatenate(
        [jnp.zeros((T, T), f32), R, jnp.zeros((ZW - T - H, T), f32)],
        axis=0)
    WeC = We.transpose(1, 0, 2).reshape(T, NE * T)
    R4 = jnp.kron(jnp.eye(NE, dtype=f32), jnp.ones((1, T), f32))
    K4 = jnp.tile(jnp.eye(T, dtype=f32), (NE, 1))
    zz = jnp.zeros((N, ZW), f32)

    gathered = _sc_gather(src_embedding, src)
    z = _tc_edge(gathered, edge_embedding, W_fc, b_fc.reshape(1, T), Wv, A, R,
                 S64)
    p0, p1 = _sc_scatter(z, dst, zz)
    return _tc_node(p0, p1, M, Wo, Wg, WeC, R4, K4)


# speedup vs baseline: 61.4855x; 1.2232x over previous
import math

import jax
import jax.numpy as jnp
from jax import lax
from jax.experimental import pallas as pl
from jax.experimental.pallas import tpu as pltpu
from jax.experimental.pallas import tpu_sc as plsc

N = 10000
E = 320000
D = 128
T = 64
H = 4
DH = 16
NE = 4

NC = 2
NS = 16
NW = NC * NS

CB = 80
EW = E // NW
NI = EW // CB
GF = 5
NO = NI // GF
SCB = 80
SNI = EW // SCB
SGF = 1
SNO = SNI // SGF
ZW = 128

BE = 1600
BN = 1000


def _gather_body(table_hbm, src_hbm, out_hbm, idx_all, rows_v,
                 idx_sem, gat_sem, out_sem):
    cid = lax.axis_index("c")
    sid = lax.axis_index("s")
    wid = sid * NC + cid
    wbase = wid * EW

    pltpu.async_copy(src_hbm.at[pl.ds(wbase, EW)], idx_all, idx_sem).wait()

    def outer(o, carry):
        g = lax.rem(o, 2)
        s0 = g * GF

        @pl.when(o >= 1)
        def _():
            for b in range(GF):
                pltpu.make_async_copy(
                    rows_v.at[b], out_hbm.at[pl.ds(wbase, CB)], out_sem).wait()
        for b in range(GF):
            ci = o * GF + b
            pltpu.async_copy(
                table_hbm.at[idx_all.at[pl.ds(ci * CB, CB)]],
                rows_v.at[s0 + b], gat_sem)
        for b in range(GF):
            pltpu.make_async_copy(
                table_hbm.at[idx_all.at[pl.ds(0, CB)]], rows_v.at[s0 + b],
                gat_sem).wait()
        for b in range(GF):
            ci = o * GF + b
            pltpu.async_copy(rows_v.at[s0 + b],
                             out_hbm.at[pl.ds(wbase + ci * CB, CB)], out_sem)
        return carry

    lax.fori_loop(0, NO, outer, 0)
    for b in range(GF):
        pltpu.make_async_copy(
            rows_v.at[b], out_hbm.at[pl.ds(wbase, CB)], out_sem).wait()


def _sc_gather(table, src):
    mesh = plsc.VectorSubcoreMesh(core_axis_name="c", subcore_axis_name="s")
    f = pl.kernel(
        _gather_body,
        out_type=jax.ShapeDtypeStruct((E, D), jnp.float32),
        mesh=mesh,
        scratch_types=[
            pltpu.VMEM((EW,), jnp.int32),
            pltpu.VMEM((2 * GF, CB, D), jnp.float32),
            pltpu.SemaphoreType.DMA,
            pltpu.SemaphoreType.DMA,
            pltpu.SemaphoreType.DMA,
        ],
    )
    return f(table, src)


def _edge_body(g_ref, e_ref, wfc_ref, bfc_ref, wv_ref, a_ref, r_ref, s64_ref,
               z_ref):
    msg = jnp.maximum(g_ref[...] + e_ref[...], 0.0)
    p = jnp.dot(msg, wfc_ref[...], preferred_element_type=jnp.float32)
    p = p + bfc_ref[...]
    v = jnp.dot(p, wv_ref[...], preferred_element_type=jnp.float32)
    s = jnp.clip(jnp.dot(p, a_ref[...], preferred_element_type=jnp.float32),
                 -60.0, 60.0)
    ex = jnp.exp(s)
    exrep = jnp.dot(ex, r_ref[...], preferred_element_type=jnp.float32)
    vex = v * exrep
    extail = jnp.dot(ex, s64_ref[...], preferred_element_type=jnp.float32)
    z_ref[...] = jnp.concatenate([vex, extail], axis=1)


def _tc_edge(gathered, edge_emb, wfc, bfc, wv, a, r, s64):
    return pl.pallas_call(
        _edge_body,
        grid=(E // BE,),
        in_specs=[
            pl.BlockSpec((BE, D), lambda i: (i, 0)),
            pl.BlockSpec((BE, D), lambda i: (i, 0)),
            pl.BlockSpec((D, T), lambda i: (0, 0)),
            pl.BlockSpec((1, T), lambda i: (0, 0)),
            pl.BlockSpec((T, T), lambda i: (0, 0)),
            pl.BlockSpec((T, H), lambda i: (0, 0)),
            pl.BlockSpec((H, T), lambda i: (0, 0)),
            pl.BlockSpec((H, T), lambda i: (0, 0)),
        ],
        out_specs=pl.BlockSpec((BE, ZW), lambda i: (i, 0)),
        out_shape=jax.ShapeDtypeStruct((E, ZW), jnp.float32),
    )(gathered, edge_emb, wfc, bfc, wv, a, r, s64)


def _scatter_body(z_hbm, dst2_hbm, zz_hbm, out0_hbm, out1_hbm,
                  idx2_v, z_v, acc_sh, ld_sem, sc_sem):
    cid = lax.axis_index("c")
    sid = lax.axis_index("s")
    wid = sid * NC + cid
    wbase = wid * EW

    pltpu.async_copy(dst2_hbm.at[wid], idx2_v, ld_sem).wait()

    @pl.when(sid == 0)
    def _():
        pltpu.sync_copy(zz_hbm, acc_sh)

    plsc.subcore_barrier()

    for b in range(SGF):
        pltpu.async_copy(z_hbm.at[pl.ds(wbase + b * SCB, SCB)], z_v.at[b],
                         ld_sem)

    def outer(o, carry):
        g = lax.rem(o, 2)
        s0 = g * SGF

        @pl.when(o >= 1)
        def _():
            for b in range(SGF):
                pltpu.make_async_copy(
                    z_v.at[b], acc_sh.at[idx2_v.at[0]], sc_sem).wait()
        for b in range(SGF):
            pltpu.make_async_copy(
                z_hbm.at[pl.ds(wbase, SCB)], z_v.at[b], ld_sem).wait()
        for b in range(SGF):
            ci = o * SGF + b
            pltpu.async_copy(z_v.at[s0 + b], acc_sh.at[idx2_v.at[ci]],
                             sc_sem, add=True)

        @pl.when(o < SNO - 1)
        def _():
            for b in range(SGF):
                ci = (o + 1) * SGF + b
                pltpu.async_copy(z_hbm.at[pl.ds(wbase + ci * SCB, SCB)],
                                 z_v.at[(1 - g) * SGF + b], ld_sem)
        return carry

    lax.fori_loop(0, SNO, outer, 0)
    for b in range(SGF):
        pltpu.make_async_copy(
            z_v.at[b], acc_sh.at[idx2_v.at[0]], sc_sem).wait()

    plsc.subcore_barrier()

    @pl.when(jnp.logical_and(sid == 0, cid == 0))
    def _():
        pltpu.sync_copy(acc_sh, out0_hbm)

    @pl.when(jnp.logical_and(sid == 0, cid == 1))
    def _():
        pltpu.sync_copy(acc_sh, out1_hbm)


def _sc_scatter(z, dst, zz):
    mesh = plsc.VectorSubcoreMesh(core_axis_name="c", subcore_axis_name="s")
    f = pl.kernel(
        _scatter_body,
        out_type=(jax.ShapeDtypeStruct((N, ZW), jnp.float32),
                  jax.ShapeDtypeStruct((N, ZW), jnp.float32)),
        mesh=mesh,
        scratch_types=[
            pltpu.VMEM((SNI, SCB), jnp.int32),
            pltpu.VMEM((2 * SGF, SCB, ZW), jnp.float32),
            pltpu.VMEM_SHARED((N, ZW), jnp.float32),
            pltpu.SemaphoreType.DMA,
            pltpu.SemaphoreType.DMA,
        ],
    )
    return f(z, dst.reshape(NW, SNI, SCB), zz)


def _node_body(p0_ref, p1_ref, m_ref, wo_ref, wg_ref, wec_ref, r4_ref, k4_ref,
               out_ref):
    acc = p0_ref[...] + p1_ref[...]
    denrep = jnp.dot(acc, m_ref[...], preferred_element_type=jnp.float32)
    vex = acc[:, :T]
    agg = vex / (denrep + 1e-9)
    attn = jnp.dot(agg, wo_ref[...], preferred_element_type=jnp.float32)
    gl = jnp.dot(attn, wg_ref[...], preferred_element_type=jnp.float32)
    gm = jnp.max(gl, axis=1, keepdims=True)
    ge = jnp.exp(gl - gm)
    gate = ge / jnp.sum(ge, axis=1, keepdims=True)
    expf = jnp.dot(attn, wec_ref[...], preferred_element_type=jnp.float32)
    grep = jnp.dot(gate, r4_ref[...], preferred_element_type=jnp.float32)
    moe = jnp.dot(grep * expf, k4_ref[...], preferred_element_type=jnp.float32)
    out_ref[...] = 1.0 / (1.0 + jnp.exp(-moe))


def _tc_node(p0, p1, m, wo, wg, wec, r4, k4):
    return pl.pallas_call(
        _node_body,
        grid=(N // BN,),
        in_specs=[
            pl.BlockSpec((BN, ZW), lambda i: (i, 0)),
            pl.BlockSpec((BN, ZW), lambda i: (i, 0)),
            pl.BlockSpec((ZW, T), lambda i: (0, 0)),
            pl.BlockSpec((T, T), lambda i: (0, 0)),
            pl.BlockSpec((T, NE), lambda i: (0, 0)),
            pl.BlockSpec((T, NE * T), lambda i: (0, 0)),
            pl.BlockSpec((NE, NE * T), lambda i: (0, 0)),
            pl.BlockSpec((NE * T, T), lambda i: (0, 0)),
        ],
        out_specs=pl.BlockSpec((BN, T), lambda i: (i, 0)),
        out_shape=jax.ShapeDtypeStruct((N, T), jnp.float32),
    )(p0, p1, m, wo, wg, wec, r4, k4)


def kernel(src_embedding, edge_index, edge_embedding, W_fc, b_fc, q, Wk, Wv,
           Wo, Wg, We):
    f32 = jnp.float32
    src = edge_index[0].astype(jnp.int32)
    dst = edge_index[1].astype(jnp.int32)

    A = (Wk.reshape(T, H, DH) * q[None, :, :]).sum(-1) * (1.0 / math.sqrt(DH))
    R = jnp.kron(jnp.eye(H, dtype=f32), jnp.ones((1, DH), f32))
    S64 = jnp.eye(H, T, dtype=f32)
    M = jnp.concatenate(
        [jnp.zeros((T, T), f32), R, jnp.zeros((ZW - T - H, T), f32)],
        axis=0)
    WeC = We.transpose(1, 0, 2).reshape(T, NE * T)
    R4 = jnp.kron(jnp.eye(NE, dtype=f32), jnp.ones((1, T), f32))
    K4 = jnp.tile(jnp.eye(T, dtype=f32), (NE, 1))
    zz = jnp.zeros((N, ZW), f32)

    gathered = _sc_gather(src_embedding, src)
    z = _tc_edge(gathered, edge_embedding, W_fc, b_fc.reshape(1, T), Wv, A, R,
                 S64)
    p0, p1 = _sc_scatter(z, dst, zz)
    return _tc_node(p0, p1, M, Wo, Wg, WeC, R4, K4)
